# baseline (speedup 1.0000x reference)
import jax
import jax.numpy as jnp
from jax import lax
from jax.experimental import pallas as pl
from jax.experimental.pallas import tpu as pltpu

D = 128
IN_DEG = 4
NUM_T = 4
CHUNK = 1024
TILE = 128


def _gelu_exact(x):
    return x * 0.5 * (1.0 + lax.erf(x * 0.7071067811865476))


def _body(idx_ref, t_ref, root_ref, W1_ref, b1_ref, W2_ref, b2_ref,
          out_ref, lev_ref, bin_ref, order_ref, X0_ref, X1_ref, X2_ref,
          X3_ref, E_ref):
    X_refs = (X0_ref, X1_ref, X2_ref, X3_ref)
    R = root_ref.shape[0]
    N = t_ref.shape[0]
    out_ref[0:R, :] = root_ref[...]

    def lev_root(i, c):
        lev_ref[i] = 0
        return c
    lax.fori_loop(0, R, lev_root, 0)

    def zero_init(k, c):
        bin_ref[k] = 0
        return c
    lax.fori_loop(0, 2 * NUM_T, zero_init, 0)

    def lev_step(i, lmax):
        m = lev_ref[idx_ref[i * IN_DEG + 0]]
        for j in range(1, IN_DEG):
            m = jnp.maximum(m, lev_ref[idx_ref[i * IN_DEG + j]])
        l = m + 1
        lev_ref[R + i] = l

        @pl.when(l > lmax)
        def _():
            for u in range(NUM_T):
                bin_ref[(l + 1) * NUM_T + u] = 0

        k = l * NUM_T + t_ref[i]
        bin_ref[k] = bin_ref[k] + 1
        return jnp.maximum(lmax, l)
    lev_max = lax.fori_loop(0, N, lev_step, jnp.int32(0))
    nbins = (lev_max + 1) * NUM_T

    def pfx_step(k, run):
        v = bin_ref[k]
        bin_ref[k] = run
        return run + v
    lax.fori_loop(0, nbins, pfx_step, jnp.int32(0))

    def place_step(i, c):
        k = lev_ref[R + i] * NUM_T + t_ref[i]
        p = bin_ref[k]
        order_ref[p] = i
        bin_ref[k] = p + 1
        return c
    lax.fori_loop(0, N, place_step, 0)

    def level_body(l, c):
        for t in range(NUM_T):
            k = l * NUM_T + t
            start = bin_ref[k - 1]
            end = bin_ref[k]
            cnt = end - start
            nchunks = (cnt + CHUNK - 1) // CHUNK

            def chunk_body(cix, c2):
                base = start + cix * CHUNK
                clen = jnp.minimum(CHUNK, cnt - cix * CHUNK)

                npairs = clen // 2

                def gather_pair(r2, c3):
                    r = pl.multiple_of(r2 * 2, 2)
                    na = order_ref[base + r]
                    nb = order_ref[base + r + 1]
                    for j in range(IN_DEG):
                        pa = idx_ref[na * IN_DEG + j]
                        pb = idx_ref[nb * IN_DEG + j]
                        ra = out_ref[pl.ds(pa, 1), :]
                        rb = out_ref[pl.ds(pb, 1), :]
                        X_refs[j][pl.ds(r, 2), :] = \
                            jnp.concatenate([ra, rb], axis=0)
                    return c3
                lax.fori_loop(0, npairs, gather_pair, 0)

                def gather_body(r, c3):
                    node = order_ref[base + r]
                    for j in range(IN_DEG):
                        p = idx_ref[node * IN_DEG + j]
                        X_refs[j][pl.ds(r, 1), :] = out_ref[pl.ds(p, 1), :]
                    return c3
                lax.fori_loop(npairs * 2, clen, gather_body, 0)

                ntiles = (clen + TILE - 1) // TILE

                def tile_body(ti, c3):
                    h = b1_ref[t:t + 1, :]
                    for j in range(IN_DEG):
                        xj = X_refs[j][pl.ds(ti * TILE, TILE), :]
                        h = h + jnp.dot(
                            xj, W1_ref[t, pl.ds(j * D, D), :],
                            preferred_element_type=jnp.float32)
                    g = _gelu_exact(h)
                    e = jnp.dot(g, W2_ref[t],
                                preferred_element_type=jnp.float32)
                    e = e + b2_ref[t:t + 1, :]
                    E_ref[pl.ds(ti * TILE, TILE), :] = e
                    return c3
                lax.fori_loop(0, ntiles, tile_body, 0)

                def scat_body(r, c3):
                    node = order_ref[base + r]
                    out_ref[pl.ds(R + node, 1), :] = E_ref[pl.ds(r, 1), :]
                    return c3
                lax.fori_loop(0, clen, scat_body, 0)
                return c2
            lax.fori_loop(0, nchunks, chunk_body, 0)
        return c
    lax.fori_loop(1, lev_max + 1, level_body, 0)


def kernel(root_emb, W1, b1, W2, b2, trunk_node_inputs_indices, trunk_node_types):
    R = root_emb.shape[0]
    N = trunk_node_inputs_indices.shape[0]
    idx32 = trunk_node_inputs_indices.astype(jnp.int32).reshape(-1)
    t32 = trunk_node_types.astype(jnp.int32)
    out = pl.pallas_call(
        _body,
        out_shape=jax.ShapeDtypeStruct((R + N, D), jnp.float32),
        in_specs=[
            pl.BlockSpec(memory_space=pltpu.SMEM),
            pl.BlockSpec(memory_space=pltpu.SMEM),
            pl.BlockSpec(memory_space=pltpu.VMEM),
            pl.BlockSpec(memory_space=pltpu.VMEM),
            pl.BlockSpec(memory_space=pltpu.VMEM),
            pl.BlockSpec(memory_space=pltpu.VMEM),
            pl.BlockSpec(memory_space=pltpu.VMEM),
        ],
        out_specs=pl.BlockSpec(memory_space=pltpu.VMEM),
        scratch_shapes=[
            pltpu.SMEM((R + N,), jnp.int32),
            pltpu.SMEM((NUM_T * (N + 2),), jnp.int32),
            pltpu.SMEM((N,), jnp.int32),
            pltpu.VMEM((CHUNK, D), jnp.float32),
            pltpu.VMEM((CHUNK, D), jnp.float32),
            pltpu.VMEM((CHUNK, D), jnp.float32),
            pltpu.VMEM((CHUNK, D), jnp.float32),
            pltpu.VMEM((CHUNK, D), jnp.float32),
        ],
    )(idx32, t32, root_emb, W1, b1, W2, b2)
    return out


# speedup vs baseline: 1.3728x; 1.3728x over previous
import jax
import jax.numpy as jnp
from jax import lax
from jax.experimental import pallas as pl
from jax.experimental.pallas import tpu as pltpu

D = 128
IN_DEG = 4
NUM_T = 4
CHUNK = 1024
TILE = 128


def _gelu_exact(x):
    return x * 0.5 * (1.0 + lax.erf(x * 0.7071067811865476))


def _body(idx_ref, t_ref, root_ref, W1_ref, b1_ref, W2_ref, b2_ref,
          out_ref, lev_ref, bin_ref, order_ref, X0_ref, X1_ref, X2_ref,
          X3_ref, E_ref):
    X_refs = (X0_ref, X1_ref, X2_ref, X3_ref)
    R = root_ref.shape[0]
    N = t_ref.shape[0]
    out_ref[0:R, :] = root_ref[...]

    def lev_root(i, c):
        lev_ref[i] = 0
        return c
    lax.fori_loop(0, R, lev_root, 0)

    def zero_init(k, c):
        bin_ref[k] = 0
        return c
    lax.fori_loop(0, 2 * NUM_T, zero_init, 0)

    def lev_step(i, lmax):
        m = lev_ref[idx_ref[i * IN_DEG + 0]]
        for j in range(1, IN_DEG):
            m = jnp.maximum(m, lev_ref[idx_ref[i * IN_DEG + j]])
        l = m + 1
        lev_ref[R + i] = l

        @pl.when(l > lmax)
        def _():
            for u in range(NUM_T):
                bin_ref[(l + 1) * NUM_T + u] = 0

        k = l * NUM_T + t_ref[i]
        bin_ref[k] = bin_ref[k] + 1
        return jnp.maximum(lmax, l)
    lev_max = lax.fori_loop(0, N, lev_step, jnp.int32(0))
    nbins = (lev_max + 1) * NUM_T

    def pfx_step(k, run):
        v = bin_ref[k]
        bin_ref[k] = run
        return run + v
    lax.fori_loop(0, nbins, pfx_step, jnp.int32(0))

    def place_step(i, c):
        k = lev_ref[R + i] * NUM_T + t_ref[i]
        p = bin_ref[k]
        order_ref[p] = i
        bin_ref[k] = p + 1
        return c
    lax.fori_loop(0, N, place_step, 0)

    def level_body(l, c):
        for t in range(NUM_T):
            k = l * NUM_T + t
            start = bin_ref[k - 1]
            end = bin_ref[k]
            cnt = end - start
            nchunks = (cnt + CHUNK - 1) // CHUNK

            def chunk_body(cix, c2):
                base = start + cix * CHUNK
                clen = jnp.minimum(CHUNK, cnt - cix * CHUNK)

                def gather_body(r, c3):
                    node = order_ref[base + r]
                    for j in range(IN_DEG):
                        p = idx_ref[node * IN_DEG + j]
                        X_refs[j][pl.ds(r, 1), :] = out_ref[pl.ds(p, 1), :]
                    return c3
                lax.fori_loop(0, clen, gather_body, 0)

                ntiles = (clen + TILE - 1) // TILE

                def tile_body(ti, c3):
                    h = b1_ref[t:t + 1, :]
                    for j in range(IN_DEG):
                        xj = X_refs[j][pl.ds(ti * TILE, TILE), :]
                        h = h + jnp.dot(
                            xj, W1_ref[t, pl.ds(j * D, D), :],
                            preferred_element_type=jnp.float32)
                    g = _gelu_exact(h)
                    e = jnp.dot(g, W2_ref[t],
                                preferred_element_type=jnp.float32)
                    e = e + b2_ref[t:t + 1, :]
                    E_ref[pl.ds(ti * TILE, TILE), :] = e
                    return c3
                lax.fori_loop(0, ntiles, tile_body, 0)

                nsp = clen // 2

                def scat_pair(r2, c3):
                    r = pl.multiple_of(r2 * 2, 2)
                    e2 = E_ref[pl.ds(r, 2), :]
                    na = order_ref[base + r]
                    nb = order_ref[base + r + 1]
                    out_ref[pl.ds(R + na, 1), :] = e2[0:1, :]
                    out_ref[pl.ds(R + nb, 1), :] = e2[1:2, :]
                    return c3
                lax.fori_loop(0, nsp, scat_pair, 0)

                def scat_body(r, c3):
                    node = order_ref[base + r]
                    out_ref[pl.ds(R + node, 1), :] = E_ref[pl.ds(r, 1), :]
                    return c3
                lax.fori_loop(nsp * 2, clen, scat_body, 0)
                return c2
            lax.fori_loop(0, nchunks, chunk_body, 0)
        return c
    lax.fori_loop(1, lev_max + 1, level_body, 0)


def kernel(root_emb, W1, b1, W2, b2, trunk_node_inputs_indices, trunk_node_types):
    R = root_emb.shape[0]
    N = trunk_node_inputs_indices.shape[0]
    idx32 = trunk_node_inputs_indices.astype(jnp.int32).reshape(-1)
    t32 = trunk_node_types.astype(jnp.int32)
    out = pl.pallas_call(
        _body,
        out_shape=jax.ShapeDtypeStruct((R + N, D), jnp.float32),
        in_specs=[
            pl.BlockSpec(memory_space=pltpu.SMEM),
            pl.BlockSpec(memory_space=pltpu.SMEM),
            pl.BlockSpec(memory_space=pltpu.VMEM),
            pl.BlockSpec(memory_space=pltpu.VMEM),
            pl.BlockSpec(memory_space=pltpu.VMEM),
            pl.BlockSpec(memory_space=pltpu.VMEM),
            pl.BlockSpec(memory_space=pltpu.VMEM),
        ],
        out_specs=pl.BlockSpec(memory_space=pltpu.VMEM),
        scratch_shapes=[
            pltpu.SMEM((R + N,), jnp.int32),
            pltpu.SMEM((NUM_T * (N + 2),), jnp.int32),
            pltpu.SMEM((N,), jnp.int32),
            pltpu.VMEM((CHUNK, D), jnp.float32),
            pltpu.VMEM((CHUNK, D), jnp.float32),
            pltpu.VMEM((CHUNK, D), jnp.float32),
            pltpu.VMEM((CHUNK, D), jnp.float32),
            pltpu.VMEM((CHUNK, D), jnp.float32),
        ],
    )(idx32, t32, root_emb, W1, b1, W2, b2)
    return out


# speedup vs baseline: 1.4028x; 1.0219x over previous
import jax
import jax.numpy as jnp
from jax import lax
from jax.experimental import pallas as pl
from jax.experimental.pallas import tpu as pltpu

D = 128
IN_DEG = 4
NUM_T = 4
CHUNK = 1024
TILE = 128


def _gelu_exact(x):
    return x * 0.5 * (1.0 + lax.erf(x * 0.7071067811865476))


def _body(idx_ref, t_ref, root_ref, W1_ref, b1_ref, W2_ref, b2_ref,
          out_ref, lev_ref, bin_ref, order_ref, X0_ref, X1_ref, X2_ref,
          X3_ref, E_ref):
    X_refs = (X0_ref, X1_ref, X2_ref, X3_ref)
    R = root_ref.shape[0]
    N = t_ref.shape[0]
    out_ref[0:R, :] = root_ref[...]

    def lev_root(i, c):
        lev_ref[i] = 0
        return c
    lax.fori_loop(0, R, lev_root, 0)

    def zero_init(k, c):
        bin_ref[k] = 0
        return c
    lax.fori_loop(0, 2 * NUM_T, zero_init, 0)

    def lev_step(i, lmax):
        m = lev_ref[idx_ref[i * IN_DEG + 0]]
        for j in range(1, IN_DEG):
            m = jnp.maximum(m, lev_ref[idx_ref[i * IN_DEG + j]])
        l = m + 1
        lev_ref[R + i] = l

        @pl.when(l > lmax)
        def _():
            for u in range(NUM_T):
                bin_ref[(l + 1) * NUM_T + u] = 0

        k = l * NUM_T + t_ref[i]
        bin_ref[k] = bin_ref[k] + 1
        return jnp.maximum(lmax, l)
    lev_max = lax.fori_loop(0, N, lev_step, jnp.int32(0))
    nbins = (lev_max + 1) * NUM_T

    def pfx_step(k, run):
        v = bin_ref[k]
        bin_ref[k] = run
        return run + v
    lax.fori_loop(0, nbins, pfx_step, jnp.int32(0))

    def place_step(i, c):
        k = lev_ref[R + i] * NUM_T + t_ref[i]
        p = bin_ref[k]
        order_ref[p] = i
        bin_ref[k] = p + 1
        return c
    lax.fori_loop(0, N, place_step, 0)

    def level_body(l, c):
        for t in range(NUM_T):
            k = l * NUM_T + t
            start = bin_ref[k - 1]
            end = bin_ref[k]
            cnt = end - start
            nchunks = (cnt + CHUNK - 1) // CHUNK

            def chunk_body(cix, c2):
                base = start + cix * CHUNK
                clen = jnp.minimum(CHUNK, cnt - cix * CHUNK)

                def gather_body(r, c3):
                    node = order_ref[base + r]
                    for j in range(IN_DEG):
                        p = idx_ref[node * IN_DEG + j]
                        X_refs[j][pl.ds(r, 1), :] = out_ref[pl.ds(p, 1), :]
                    return c3
                lax.fori_loop(0, clen, gather_body, 0)

                ntiles = (clen + TILE - 1) // TILE

                def tile_body(ti, c3):
                    h = b1_ref[t:t + 1, :]
                    for j in range(IN_DEG):
                        xj = X_refs[j][pl.ds(ti * TILE, TILE), :]
                        h = h + jnp.dot(
                            xj, W1_ref[t, pl.ds(j * D, D), :],
                            preferred_element_type=jnp.float32)
                    g = _gelu_exact(h)
                    e = jnp.dot(g, W2_ref[t],
                                preferred_element_type=jnp.float32)
                    e = e + b2_ref[t:t + 1, :]
                    E_ref[pl.ds(ti * TILE, TILE), :] = e
                    return c3
                lax.fori_loop(0, ntiles, tile_body, 0)

                nsp = clen // 4

                def scat_quad(r4, c3):
                    r = pl.multiple_of(r4 * 4, 4)
                    e4 = E_ref[pl.ds(r, 4), :]
                    for u in range(4):
                        node = order_ref[base + r + u]
                        out_ref[pl.ds(R + node, 1), :] = e4[u:u + 1, :]
                    return c3
                lax.fori_loop(0, nsp, scat_quad, 0)

                def scat_body(r, c3):
                    node = order_ref[base + r]
                    out_ref[pl.ds(R + node, 1), :] = E_ref[pl.ds(r, 1), :]
                    return c3
                lax.fori_loop(nsp * 4, clen, scat_body, 0)
                return c2
            lax.fori_loop(0, nchunks, chunk_body, 0)
        return c
    lax.fori_loop(1, lev_max + 1, level_body, 0)


def kernel(root_emb, W1, b1, W2, b2, trunk_node_inputs_indices, trunk_node_types):
    R = root_emb.shape[0]
    N = trunk_node_inputs_indices.shape[0]
    idx32 = trunk_node_inputs_indices.astype(jnp.int32).reshape(-1)
    t32 = trunk_node_types.astype(jnp.int32)
    out = pl.pallas_call(
        _body,
        out_shape=jax.ShapeDtypeStruct((R + N, D), jnp.float32),
        in_specs=[
            pl.BlockSpec(memory_space=pltpu.SMEM),
            pl.BlockSpec(memory_space=pltpu.SMEM),
            pl.BlockSpec(memory_space=pltpu.VMEM),
            pl.BlockSpec(memory_space=pltpu.VMEM),
            pl.BlockSpec(memory_space=pltpu.VMEM),
            pl.BlockSpec(memory_space=pltpu.VMEM),
            pl.BlockSpec(memory_space=pltpu.VMEM),
        ],
        out_specs=pl.BlockSpec(memory_space=pltpu.VMEM),
        scratch_shapes=[
            pltpu.SMEM((R + N,), jnp.int32),
            pltpu.SMEM((NUM_T * (N + 2),), jnp.int32),
            pltpu.SMEM((N,), jnp.int32),
            pltpu.VMEM((CHUNK, D), jnp.float32),
            pltpu.VMEM((CHUNK, D), jnp.float32),
            pltpu.VMEM((CHUNK, D), jnp.float32),
            pltpu.VMEM((CHUNK, D), jnp.float32),
            pltpu.VMEM((CHUNK, D), jnp.float32),
        ],
    )(idx32, t32, root_emb, W1, b1, W2, b2)
    return out
